# Initial kernel scaffold; baseline (speedup 1.0000x reference)
import jax
import jax.numpy as jnp
from jax.experimental import pallas as pl


def kernel(x, edge_index, Wl0, bl0, Wr0, Wl1, bl1, Wr1, Wl2, bl2, Wr2):
    raise NotImplementedError("write your pallas kernel here")


# speedup vs baseline: 10.2828x; 10.2828x over previous
import functools

import jax
import jax.numpy as jnp
from jax import lax
from jax.experimental import pallas as pl
from jax.experimental.pallas import tpu as pltpu, tpu_sc as plsc

NC, NS = 2, 16
NW = NC * NS
CH = 128
CW = 16


def _tc_proj(x_ref, wlT_ref, wrT_ref, bl_ref, z_ref, r_ref):
    xv = x_ref[...]
    z_ref[...] = jnp.dot(xv, wlT_ref[...], preferred_element_type=jnp.float32)
    r_ref[...] = (jnp.dot(xv, wrT_ref[...], preferred_element_type=jnp.float32)
                  + bl_ref[...])


def _tc_mid(part_ref, cntp_ref, r_ref, wlT_ref, wrT_ref, bl_ref, z_ref, rn_ref):
    n = r_ref.shape[0]
    cnt = cntp_ref[0, :n, 0:1] + cntp_ref[1, :n, 0:1]
    inv = 1.0 / jnp.maximum(cnt, 1.0)
    agg = part_ref[0, :n, :] + part_ref[1, :n, :]
    h = jnp.maximum(agg * inv + r_ref[...], 0.0)
    z_ref[...] = jnp.dot(h, wlT_ref[...], preferred_element_type=jnp.float32)
    rn_ref[...] = (jnp.dot(h, wrT_ref[...], preferred_element_type=jnp.float32)
                   + bl_ref[...])


def _tc_fin(part_ref, cntp_ref, r_ref, o_ref):
    n = r_ref.shape[0]
    cnt = cntp_ref[0, :n, 0:1] + cntp_ref[1, :n, 0:1]
    inv = 1.0 / jnp.maximum(cnt, 1.0)
    agg = part_ref[0, :n, :] + part_ref[1, :n, :]
    o_ref[...] = agg * inv + r_ref[...]


def _sc_agg(n, npad, k, dh, with_cnt, *refs):
    if with_cnt:
        (z_hbm, src_hbm, dst_hbm, ones_hbm, zeros_hbm, zerosc_hbm,
         part_hbm, cntp_hbm,
         src_v, dst_v, rows_v, ones_v, acc_sh, cnt_sh, sem) = refs
    else:
        (z_hbm, src_hbm, dst_hbm, zeros_hbm,
         part_hbm,
         src_v, dst_v, rows_v, acc_sh, sem) = refs
    c = lax.axis_index("c")
    s = lax.axis_index("s")
    blk = c * NS + s
    zrows = npad // NS
    zr = s * zrows
    pltpu.sync_copy(zeros_hbm.at[pl.ds(zr, zrows)], acc_sh.at[pl.ds(zr, zrows)])
    if with_cnt:
        pltpu.sync_copy(zerosc_hbm.at[pl.ds(zr, zrows)],
                        cnt_sh.at[pl.ds(zr, zrows)])
        pltpu.sync_copy(ones_hbm, ones_v)
    pltpu.sync_copy(src_hbm.at[blk], src_v)
    pltpu.sync_copy(dst_hbm.at[blk], dst_v)
    plsc.subcore_barrier()

    def body(j, carry):
        pltpu.async_copy(z_hbm.at[src_v.at[j]], rows_v, sem).wait()
        pltpu.sync_copy(rows_v, acc_sh.at[dst_v.at[j]], add=True)
        if with_cnt:
            pltpu.sync_copy(ones_v, cnt_sh.at[dst_v.at[j]], add=True)
        return carry

    lax.fori_loop(0, k, body, 0)
    plsc.subcore_barrier()
    pltpu.sync_copy(acc_sh.at[pl.ds(zr, zrows)],
                    part_hbm.at[c, pl.ds(zr, zrows)])
    if with_cnt:
        pltpu.sync_copy(cnt_sh.at[pl.ds(zr, zrows)],
                        cntp_hbm.at[c, pl.ds(zr, zrows)])


def kernel(x, edge_index, Wl0, bl0, Wr0, Wl1, bl1, Wr1, Wl2, bl2, Wr2):
    n, d_in = x.shape
    dh = Wl0.shape[0]
    e = edge_index.shape[1]
    k = -(-e // (NW * CH))
    epad = NW * k * CH
    npad = NS * (-(-(n + 1) // (NS * 8)) * 8)

    src = edge_index[0]
    dst = edge_index[1]
    pad = epad - e
    srcp = jnp.concatenate([src, jnp.zeros((pad,), jnp.int32)]).reshape(NW, k, CH)
    dstp = jnp.concatenate([dst, jnp.full((pad,), n, jnp.int32)]).reshape(NW, k, CH)
    ones = jnp.ones((CH, CW), jnp.float32)
    zeros = jnp.zeros((npad, dh), jnp.float32)
    zerosc = jnp.zeros((npad, CW), jnp.float32)
    f32 = jnp.float32

    proj = pl.pallas_call(
        _tc_proj,
        out_shape=(jax.ShapeDtypeStruct((n, dh), f32),
                   jax.ShapeDtypeStruct((n, dh), f32)),
    )
    mid = pl.pallas_call(
        _tc_mid,
        out_shape=(jax.ShapeDtypeStruct((n, dh), f32),
                   jax.ShapeDtypeStruct((n, dh), f32)),
    )
    fin = pl.pallas_call(
        _tc_fin,
        out_shape=jax.ShapeDtypeStruct((n, dh), f32),
    )

    mesh = plsc.VectorSubcoreMesh(core_axis_name="c", subcore_axis_name="s",
                                  num_cores=NC, num_subcores=NS)
    sc_params = pltpu.CompilerParams(use_tc_tiling_on_sc=False)
    common_scratch = [
        pltpu.VMEM((k, CH), jnp.int32),
        pltpu.VMEM((k, CH), jnp.int32),
        pltpu.VMEM((CH, dh), f32),
    ]
    agg_cnt = pl.kernel(
        functools.partial(_sc_agg, n, npad, k, dh, True),
        out_type=(jax.ShapeDtypeStruct((NC, npad, dh), f32),
                  jax.ShapeDtypeStruct((NC, npad, CW), f32)),
        mesh=mesh,
        scratch_types=common_scratch + [
            pltpu.VMEM((CH, CW), f32),
            pltpu.VMEM_SHARED((npad, dh), f32),
            pltpu.VMEM_SHARED((npad, CW), f32),
            pltpu.SemaphoreType.DMA,
        ],
        compiler_params=sc_params,
    )
    agg = pl.kernel(
        functools.partial(_sc_agg, n, npad, k, dh, False),
        out_type=jax.ShapeDtypeStruct((NC, npad, dh), f32),
        mesh=mesh,
        scratch_types=common_scratch + [
            pltpu.VMEM_SHARED((npad, dh), f32),
            pltpu.SemaphoreType.DMA,
        ],
        compiler_params=sc_params,
    )

    bl0r = bl0.reshape(1, dh)
    bl1r = bl1.reshape(1, dh)
    bl2r = bl2.reshape(1, dh)

    z0, r0 = proj(x, Wl0.T, Wr0.T, bl0r)
    part0, cntp = agg_cnt(z0, srcp, dstp, ones, zeros, zerosc)
    z1, r1 = mid(part0, cntp, r0, Wl1.T, Wr1.T, bl1r)
    part1 = agg(z1, srcp, dstp, zeros)
    z2, r2 = mid(part1, cntp, r1, Wl2.T, Wr2.T, bl2r)
    part2 = agg(z2, srcp, dstp, zeros)
    return fin(part2, cntp, r2)
